# Initial kernel scaffold; baseline (speedup 1.0000x reference)
import jax
import jax.numpy as jnp
from jax.experimental import pallas as pl


def kernel(hidden_states, w_router, w_gate, w_up, w_down, down_embed, up_embed):
    raise NotImplementedError("write your pallas kernel here")


# speedup vs baseline: 1.6579x; 1.6579x over previous
import functools

import jax
import jax.numpy as jnp
import numpy as np
from jax.experimental import pallas as pl
from jax.experimental.pallas import tpu as pltpu

_HIDDEN = 2048
_INTER = 4096
_NUM_KEYS = 128
_TOP_K = 64
_IB = 256


def _candidate_pairs():
    pairs = [(i, j) for i in range(_TOP_K) for j in range(_TOP_K)
             if (i + 1) * (j + 1) <= _TOP_K]
    ci = np.array([p[0] for p in pairs], np.int32)
    cj = np.array([p[1] for p in pairs], np.int32)
    return ci, cj


_CAND_I, _CAND_J = _candidate_pairs()


def _mlp_body(x_ref, wr_ref, wg_ref, wu_ref, wd_ref, shared_ref, logits_ref):
    i = pl.program_id(0)

    @pl.when(i == 0)
    def _():
        logits_ref[...] = jnp.dot(x_ref[...], wr_ref[...],
                                  preferred_element_type=jnp.float32)
        shared_ref[...] = jnp.zeros_like(shared_ref)

    xb = x_ref[...]
    g = jnp.dot(xb, wg_ref[...], preferred_element_type=jnp.float32)
    u = jnp.dot(xb, wu_ref[...], preferred_element_type=jnp.float32)
    h = jax.nn.silu(g) * u
    shared_ref[...] += jnp.dot(h, wd_ref[...],
                               preferred_element_type=jnp.float32)


def _shared_mlp(x, w_router, w_gate, w_up, w_down):
    n = x.shape[0]
    ni = _INTER // _IB
    return pl.pallas_call(
        _mlp_body,
        grid=(ni,),
        in_specs=[
            pl.BlockSpec((n, _HIDDEN), lambda i: (0, 0)),
            pl.BlockSpec((_HIDDEN, 2 * _NUM_KEYS), lambda i: (0, 0)),
            pl.BlockSpec((_HIDDEN, _IB), lambda i: (0, i)),
            pl.BlockSpec((_HIDDEN, _IB), lambda i: (0, i)),
            pl.BlockSpec((_IB, _HIDDEN), lambda i: (i, 0)),
        ],
        out_specs=[
            pl.BlockSpec((n, _HIDDEN), lambda i: (0, 0)),
            pl.BlockSpec((n, 2 * _NUM_KEYS), lambda i: (0, 0)),
        ],
        out_shape=[
            jax.ShapeDtypeStruct((n, _HIDDEN), jnp.float32),
            jax.ShapeDtypeStruct((n, 2 * _NUM_KEYS), jnp.float32),
        ],
    )(x, w_router, w_gate, w_up, w_down)


def kernel(hidden_states, w_router, w_gate, w_up, w_down, down_embed, up_embed):
    bsz, seq_len, hidden_dim = hidden_states.shape
    x = hidden_states.reshape(-1, hidden_dim)
    n_tok = x.shape[0]

    shared, logits = _shared_mlp(x, w_router, w_gate, w_up, w_down)
    router_logits = logits.reshape(2, n_tok, _NUM_KEYS)

    sx, ix = jax.lax.top_k(router_logits[0], _TOP_K)
    sy, iy = jax.lax.top_k(router_logits[1], _TOP_K)

    cand = sx[:, _CAND_I] + sy[:, _CAND_J]
    scores, pos = jax.lax.top_k(cand, _TOP_K)
    ci = jnp.asarray(_CAND_I)[pos]
    cj = jnp.asarray(_CAND_J)[pos]
    xi = jnp.take_along_axis(ix, ci, axis=1)
    yi = jnp.take_along_axis(iy, cj, axis=1)
    indices = xi * _NUM_KEYS + yi
    routing_weights = jax.nn.softmax(scores, axis=-1)

    de = jnp.take(down_embed, indices, axis=0)
    ue = jnp.take(up_embed, indices, axis=0)
    experts_weights = jnp.einsum('nkd,nd->nk', de, x)
    experts_weights = jax.nn.silu(experts_weights) * routing_weights
    experts_states = jnp.einsum('nk,nkd->nd', experts_weights, ue)

    out = shared + experts_states
    return out.reshape(bsz, seq_len, -1), router_logits


# speedup vs baseline: 2.5545x; 1.5408x over previous
import functools

import jax
import jax.numpy as jnp
import numpy as np
from jax import lax
from jax.experimental import pallas as pl
from jax.experimental.pallas import tpu as pltpu
from jax.experimental.pallas import tpu_sc as plsc

_HIDDEN = 2048
_INTER = 4096
_NUM_KEYS = 128
_TOP_K = 64
_IB = 256
_N_TOK = 2048
_NW = 32
_TPW = _N_TOK // _NW
_EC = 16
_NCHUNK = _TOP_K // _EC
_DV = _HIDDEN // 16


def _candidate_pairs():
    pairs = [(i, j) for i in range(_TOP_K) for j in range(_TOP_K)
             if (i + 1) * (j + 1) <= _TOP_K]
    ci = np.array([p[0] for p in pairs], np.int32)
    cj = np.array([p[1] for p in pairs], np.int32)
    return ci, cj


_CAND_I, _CAND_J = _candidate_pairs()


def _router_body(x_ref, wr_ref, logits_ref):
    logits_ref[...] = jnp.dot(x_ref[...], wr_ref[...],
                              preferred_element_type=jnp.float32)


def _router_logits(x, w_router):
    n = x.shape[0]
    return pl.pallas_call(
        _router_body,
        out_shape=jax.ShapeDtypeStruct((n, 2 * _NUM_KEYS), jnp.float32),
    )(x, w_router)


def _mlp_body(x_ref, wg_ref, wu_ref, wd_ref, shared_ref):
    i = pl.program_id(0)

    @pl.when(i == 0)
    def _():
        shared_ref[...] = jnp.zeros_like(shared_ref)

    xb = x_ref[...]
    g = jnp.dot(xb, wg_ref[...], preferred_element_type=jnp.float32)
    u = jnp.dot(xb, wu_ref[...], preferred_element_type=jnp.float32)
    h = jax.nn.silu(g) * u
    shared_ref[...] += jnp.dot(h, wd_ref[...],
                               preferred_element_type=jnp.float32)


def _shared_mlp(x, w_gate, w_up, w_down):
    n = x.shape[0]
    ni = _INTER // _IB
    return pl.pallas_call(
        _mlp_body,
        grid=(ni,),
        in_specs=[
            pl.BlockSpec((n, _HIDDEN), lambda i: (0, 0)),
            pl.BlockSpec((_HIDDEN, _IB), lambda i: (0, i)),
            pl.BlockSpec((_HIDDEN, _IB), lambda i: (0, i)),
            pl.BlockSpec((_IB, _HIDDEN), lambda i: (i, 0)),
        ],
        out_specs=pl.BlockSpec((n, _HIDDEN), lambda i: (0, 0)),
        out_shape=jax.ShapeDtypeStruct((n, _HIDDEN), jnp.float32),
    )(x, w_gate, w_up, w_down)


def _sc_expert_body(de_hbm, ue_hbm, x_hbm, idx_hbm, rw_hbm, out_hbm,
                    x_v, idx_v, rw_v, pmat_v, w_v, acc_v,
                    row_a, row_b, sem_a, sem_b):
    wid = lax.axis_index("s") * 2 + lax.axis_index("c")
    base = wid * _TPW
    lanes = lax.iota(jnp.int32, 16)

    @pl.loop(0, _TPW)
    def _token(t):
        n = base + t
        pltpu.sync_copy(x_hbm.at[n], x_v)
        pltpu.sync_copy(idx_hbm.at[n], idx_v)
        pltpu.sync_copy(rw_hbm.at[n], rw_v)

        bufs = (row_a, row_b)
        sems = (sem_a, sem_b)

        pltpu.async_copy(de_hbm.at[idx_v[0, :]], row_a, sem_a)
        for c in range(_NCHUNK):
            if c + 1 < _NCHUNK:
                pltpu.async_copy(de_hbm.at[idx_v[c + 1, :]],
                                 bufs[(c + 1) % 2], sems[(c + 1) % 2])
            pltpu.make_async_copy(de_hbm.at[idx_v[c, :]],
                                  bufs[c % 2], sems[c % 2]).wait()
            rows = bufs[c % 2]
            for e in range(_EC):
                @pl.loop(0, _DV, init_carry=jnp.zeros((16,), jnp.float32),
                         unroll=8)
                def _dot(i, p, _e=e, _rows=rows):
                    return p + (_rows[_e, pl.ds(i * 16, 16)]
                                * x_v[pl.ds(i * 16, 16)])
                pmat_v[e, :] = _dot
            ew = jnp.zeros((16,), jnp.float32)
            for d in range(16):
                ew = ew + plsc.load_gather(
                    pmat_v, [lanes, jnp.full((16,), d, jnp.int32)])
            sig = 1.0 / (1.0 + jnp.exp(-ew))
            w_v[pl.ds(c * 16, 16)] = ew * sig * rw_v[c, :]

        pltpu.async_copy(ue_hbm.at[idx_v[0, :]], row_a, sem_a)
        for c in range(_NCHUNK):
            if c + 1 < _NCHUNK:
                pltpu.async_copy(ue_hbm.at[idx_v[c + 1, :]],
                                 bufs[(c + 1) % 2], sems[(c + 1) % 2])
            pltpu.make_async_copy(ue_hbm.at[idx_v[c, :]],
                                  bufs[c % 2], sems[c % 2]).wait()
            rows = bufs[c % 2]
            for e in range(_EC):
                wsp = plsc.load_gather(
                    w_v, [jnp.full((16,), c * _EC + e, jnp.int32)])
                if c == 0 and e == 0:
                    @pl.loop(0, _DV, unroll=8)
                    def _acc0(i, _rows=rows, _w=wsp):
                        acc_v[pl.ds(i * 16, 16)] = _w * _rows[0, pl.ds(i * 16, 16)]
                else:
                    @pl.loop(0, _DV, unroll=8)
                    def _accn(i, _e=e, _rows=rows, _w=wsp):
                        plsc.addupdate(acc_v.at[pl.ds(i * 16, 16)],
                                       _w * _rows[_e, pl.ds(i * 16, 16)])

        pltpu.sync_copy(acc_v, out_hbm.at[n])


def _sc_experts(down_embed, up_embed, x, indices, routing_weights):
    idx = indices.reshape(_N_TOK, _NCHUNK, _EC).astype(jnp.int32)
    rw = routing_weights.reshape(_N_TOK, _NCHUNK, _EC)
    mesh = plsc.VectorSubcoreMesh(core_axis_name="c", subcore_axis_name="s",
                                  num_cores=2, num_subcores=16)
    f = pl.kernel(
        _sc_expert_body,
        out_type=jax.ShapeDtypeStruct((_N_TOK, _HIDDEN), jnp.float32),
        mesh=mesh,
        compiler_params=pltpu.CompilerParams(needs_layout_passes=False),
        scratch_types=[
            pltpu.VMEM((_HIDDEN,), jnp.float32),
            pltpu.VMEM((_NCHUNK, _EC), jnp.int32),
            pltpu.VMEM((_NCHUNK, _EC), jnp.float32),
            pltpu.VMEM((_EC, 16), jnp.float32),
            pltpu.VMEM((_TOP_K,), jnp.float32),
            pltpu.VMEM((_HIDDEN,), jnp.float32),
            pltpu.VMEM((_EC, _HIDDEN), jnp.float32),
            pltpu.VMEM((_EC, _HIDDEN), jnp.float32),
            pltpu.SemaphoreType.DMA,
            pltpu.SemaphoreType.DMA,
        ],
    )
    return f(down_embed, up_embed, x, idx, rw)


def kernel(hidden_states, w_router, w_gate, w_up, w_down, down_embed, up_embed):
    bsz, seq_len, hidden_dim = hidden_states.shape
    x = hidden_states.reshape(-1, hidden_dim)
    n_tok = x.shape[0]

    logits = _router_logits(x, w_router)
    router_logits = logits.reshape(2, n_tok, _NUM_KEYS)

    sx, ix = jax.lax.top_k(router_logits[0], _TOP_K)
    sy, iy = jax.lax.top_k(router_logits[1], _TOP_K)

    cand = sx[:, _CAND_I] + sy[:, _CAND_J]
    scores, pos = jax.lax.top_k(cand, _TOP_K)
    ci = jnp.asarray(_CAND_I)[pos]
    cj = jnp.asarray(_CAND_J)[pos]
    xi = jnp.take_along_axis(ix, ci, axis=1)
    yi = jnp.take_along_axis(iy, cj, axis=1)
    indices = xi * _NUM_KEYS + yi
    routing_weights = jax.nn.softmax(scores, axis=-1)

    experts_states = _sc_experts(down_embed, up_embed, x, indices,
                                 routing_weights)
    shared = _shared_mlp(x, w_gate, w_up, w_down)

    out = shared + experts_states
    return out.reshape(bsz, seq_len, -1), router_logits


# speedup vs baseline: 2.6310x; 1.0300x over previous
import functools

import jax
import jax.numpy as jnp
import numpy as np
from jax import lax
from jax.experimental import pallas as pl
from jax.experimental.pallas import tpu as pltpu
from jax.experimental.pallas import tpu_sc as plsc

_HIDDEN = 2048
_INTER = 4096
_NUM_KEYS = 128
_TOP_K = 64
_IB = 256
_N_TOK = 2048
_NW = 32
_TPW = _N_TOK // _NW
_EC = 16
_NCHUNK = _TOP_K // _EC
_DV = _HIDDEN // 16


def _candidate_pairs():
    pairs = [(i, j) for i in range(_TOP_K) for j in range(_TOP_K)
             if (i + 1) * (j + 1) <= _TOP_K]
    ci = np.array([p[0] for p in pairs], np.int32)
    cj = np.array([p[1] for p in pairs], np.int32)
    return ci, cj


_CAND_I, _CAND_J = _candidate_pairs()


def _router_body(x_ref, wr_ref, logits_ref):
    logits_ref[...] = jnp.dot(x_ref[...], wr_ref[...],
                              preferred_element_type=jnp.float32)


def _router_logits(x, w_router):
    n = x.shape[0]
    return pl.pallas_call(
        _router_body,
        out_shape=jax.ShapeDtypeStruct((n, 2 * _NUM_KEYS), jnp.float32),
    )(x, w_router)


def _mlp_body(x_ref, wg_ref, wu_ref, wd_ref, shared_ref):
    i = pl.program_id(0)

    @pl.when(i == 0)
    def _():
        shared_ref[...] = jnp.zeros_like(shared_ref)

    xb = x_ref[...]
    g = jnp.dot(xb, wg_ref[...], preferred_element_type=jnp.float32)
    u = jnp.dot(xb, wu_ref[...], preferred_element_type=jnp.float32)
    h = jax.nn.silu(g) * u
    shared_ref[...] += jnp.dot(h, wd_ref[...],
                               preferred_element_type=jnp.float32)


def _shared_mlp(x, w_gate, w_up, w_down):
    n = x.shape[0]
    ni = _INTER // _IB
    return pl.pallas_call(
        _mlp_body,
        grid=(ni,),
        in_specs=[
            pl.BlockSpec((n, _HIDDEN), lambda i: (0, 0)),
            pl.BlockSpec((_HIDDEN, _IB), lambda i: (0, i)),
            pl.BlockSpec((_HIDDEN, _IB), lambda i: (0, i)),
            pl.BlockSpec((_IB, _HIDDEN), lambda i: (i, 0)),
        ],
        out_specs=pl.BlockSpec((n, _HIDDEN), lambda i: (0, 0)),
        out_shape=jax.ShapeDtypeStruct((n, _HIDDEN), jnp.float32),
    )(x, w_gate, w_up, w_down)


def _sc_expert_body(de_hbm, ue_hbm, x_hbm, idx_hbm, rw_hbm, out_hbm,
                    x_v, idx_v, rw_v, pmat_v, w_v, acc_v,
                    buf0, buf1, buf2, sem0, sem1, sem2):
    wid = lax.axis_index("s") * 2 + lax.axis_index("c")
    base = wid * _TPW
    lanes = lax.iota(jnp.int32, 16)
    bufs = (buf0, buf1, buf2)
    sems = (sem0, sem1, sem2)

    def gsrc(k):
        tbl = de_hbm if k < _NCHUNK else ue_hbm
        return tbl.at[idx_v[k % _NCHUNK, :]]

    @pl.loop(0, _TPW)
    def _token(t):
        n = base + t
        cx = pltpu.async_copy(x_hbm.at[n], x_v, sem0)
        ci = pltpu.async_copy(idx_hbm.at[n], idx_v, sem1)
        cr = pltpu.async_copy(rw_hbm.at[n], rw_v, sem2)
        cx.wait()
        ci.wait()
        cr.wait()

        pltpu.async_copy(gsrc(0), bufs[0], sems[0])
        pltpu.async_copy(gsrc(1), bufs[1], sems[1])
        for k in range(2 * _NCHUNK):
            if k + 2 < 2 * _NCHUNK:
                pltpu.async_copy(gsrc(k + 2), bufs[(k + 2) % 3],
                                 sems[(k + 2) % 3])
            pltpu.make_async_copy(gsrc(k), bufs[k % 3], sems[k % 3]).wait()
            rows = bufs[k % 3]
            c = k % _NCHUNK
            if k < _NCHUNK:
                zero16 = tuple(jnp.zeros((16,), jnp.float32)
                               for _ in range(_EC))

                @pl.loop(0, _HIDDEN // 256, init_carry=zero16)
                def _dt(i, ps, _rows=rows):
                    d0 = i * 256
                    xr = [x_v[pl.ds(d0 + j * 16, 16)] for j in range(16)]
                    out = []
                    for e in range(_EC):
                        p = ps[e]
                        for j in range(16):
                            p = p + _rows[e, pl.ds(d0 + j * 16, 16)] * xr[j]
                        out.append(p)
                    return tuple(out)

                for e in range(_EC):
                    pmat_v[e, :] = _dt[e]
                ew = jnp.zeros((16,), jnp.float32)
                for d in range(16):
                    ew = ew + plsc.load_gather(
                        pmat_v, [lanes, jnp.full((16,), d, jnp.int32)])
                sig = 1.0 / (1.0 + jnp.exp(-ew))
                w_v[pl.ds(c * _EC, _EC)] = ew * sig * rw_v[c, :]
            else:
                for e in range(_EC):
                    wsp = plsc.load_gather(
                        w_v, [jnp.full((16,), c * _EC + e, jnp.int32)])
                    if c == 0 and e == 0:
                        @pl.loop(0, _DV, unroll=4)
                        def _acc0(i, _rows=rows, _w=wsp):
                            acc_v[pl.ds(i * 16, 16)] = (
                                _w * _rows[0, pl.ds(i * 16, 16)])
                    else:
                        @pl.loop(0, _DV, unroll=4)
                        def _accn(i, _e=e, _rows=rows, _w=wsp):
                            plsc.addupdate(acc_v.at[pl.ds(i * 16, 16)],
                                           _w * _rows[_e, pl.ds(i * 16, 16)])

        pltpu.sync_copy(acc_v, out_hbm.at[n])


def _sc_experts(down_embed, up_embed, x, indices, routing_weights):
    idx = indices.reshape(_N_TOK, _NCHUNK, _EC).astype(jnp.int32)
    rw = routing_weights.reshape(_N_TOK, _NCHUNK, _EC)
    mesh = plsc.VectorSubcoreMesh(core_axis_name="c", subcore_axis_name="s",
                                  num_cores=2, num_subcores=16)
    f = pl.kernel(
        _sc_expert_body,
        out_type=jax.ShapeDtypeStruct((_N_TOK, _HIDDEN), jnp.float32),
        mesh=mesh,
        compiler_params=pltpu.CompilerParams(needs_layout_passes=False),
        scratch_types=[
            pltpu.VMEM((_HIDDEN,), jnp.float32),
            pltpu.VMEM((_NCHUNK, _EC), jnp.int32),
            pltpu.VMEM((_NCHUNK, _EC), jnp.float32),
            pltpu.VMEM((_EC, 16), jnp.float32),
            pltpu.VMEM((_TOP_K,), jnp.float32),
            pltpu.VMEM((_HIDDEN,), jnp.float32),
            pltpu.VMEM((_EC, _HIDDEN), jnp.float32),
            pltpu.VMEM((_EC, _HIDDEN), jnp.float32),
            pltpu.VMEM((_EC, _HIDDEN), jnp.float32),
            pltpu.SemaphoreType.DMA,
            pltpu.SemaphoreType.DMA,
            pltpu.SemaphoreType.DMA,
        ],
    )
    return f(down_embed, up_embed, x, idx, rw)


def kernel(hidden_states, w_router, w_gate, w_up, w_down, down_embed, up_embed):
    bsz, seq_len, hidden_dim = hidden_states.shape
    x = hidden_states.reshape(-1, hidden_dim)
    n_tok = x.shape[0]

    logits = _router_logits(x, w_router)
    router_logits = logits.reshape(2, n_tok, _NUM_KEYS)

    sx, ix = jax.lax.top_k(router_logits[0], _TOP_K)
    sy, iy = jax.lax.top_k(router_logits[1], _TOP_K)

    cand = sx[:, _CAND_I] + sy[:, _CAND_J]
    scores, pos = jax.lax.top_k(cand, _TOP_K)
    ci = jnp.asarray(_CAND_I)[pos]
    cj = jnp.asarray(_CAND_J)[pos]
    xi = jnp.take_along_axis(ix, ci, axis=1)
    yi = jnp.take_along_axis(iy, cj, axis=1)
    indices = xi * _NUM_KEYS + yi
    routing_weights = jax.nn.softmax(scores, axis=-1)

    experts_states = _sc_experts(down_embed, up_embed, x, indices,
                                 routing_weights)
    shared = _shared_mlp(x, w_gate, w_up, w_down)

    out = shared + experts_states
    return out.reshape(bsz, seq_len, -1), router_logits


# speedup vs baseline: 3.9169x; 1.4887x over previous
import functools

import jax
import jax.numpy as jnp
import numpy as np
from jax import lax
from jax.experimental import pallas as pl
from jax.experimental.pallas import tpu as pltpu
from jax.experimental.pallas import tpu_sc as plsc

_HIDDEN = 2048
_INTER = 4096
_NUM_KEYS = 128
_TOP_K = 64
_IB = 256
_N_TOK = 2048
_NW = 32
_TPW = _N_TOK // _NW
_EC = 16
_NCHUNK = _TOP_K // _EC
_DV = _HIDDEN // 16


def _candidate_pairs():
    pairs = [(i, j) for i in range(_TOP_K) for j in range(_TOP_K)
             if (i + 1) * (j + 1) <= _TOP_K]
    ci = np.array([p[0] for p in pairs], np.int32)
    cj = np.array([p[1] for p in pairs], np.int32)
    return ci, cj


_CAND_I, _CAND_J = _candidate_pairs()


def _router_body(x_ref, wr_ref, logits_ref):
    logits_ref[...] = jnp.dot(x_ref[...], wr_ref[...],
                              preferred_element_type=jnp.float32)


def _router_logits(x, w_router):
    n = x.shape[0]
    return pl.pallas_call(
        _router_body,
        out_shape=jax.ShapeDtypeStruct((n, 2 * _NUM_KEYS), jnp.float32),
    )(x, w_router)


def _mlp_body(x_ref, wg_ref, wu_ref, wd_ref, shared_ref):
    i = pl.program_id(0)

    @pl.when(i == 0)
    def _():
        shared_ref[...] = jnp.zeros_like(shared_ref)

    xb = x_ref[...]
    g = jnp.dot(xb, wg_ref[...], preferred_element_type=jnp.float32)
    u = jnp.dot(xb, wu_ref[...], preferred_element_type=jnp.float32)
    h = jax.nn.silu(g) * u
    shared_ref[...] += jnp.dot(h, wd_ref[...],
                               preferred_element_type=jnp.float32)


def _shared_mlp(x, w_gate, w_up, w_down):
    n = x.shape[0]
    ni = _INTER // _IB
    return pl.pallas_call(
        _mlp_body,
        grid=(ni,),
        in_specs=[
            pl.BlockSpec((n, _HIDDEN), lambda i: (0, 0)),
            pl.BlockSpec((_HIDDEN, _IB), lambda i: (0, i)),
            pl.BlockSpec((_HIDDEN, _IB), lambda i: (0, i)),
            pl.BlockSpec((_IB, _HIDDEN), lambda i: (i, 0)),
        ],
        out_specs=pl.BlockSpec((n, _HIDDEN), lambda i: (0, 0)),
        out_shape=jax.ShapeDtypeStruct((n, _HIDDEN), jnp.float32),
    )(x, w_gate, w_up, w_down)


def _sc_expert_body(de_hbm, ue_hbm, x_hbm, idx_hbm, rw_hbm, out_hbm,
                    x_v, idx_v, rw_v, pmat_v, w_v, acc_v,
                    buf0, buf1, buf2, sem0, sem1, sem2):
    wid = lax.axis_index("s") * 2 + lax.axis_index("c")
    base = wid * _TPW
    lanes = lax.iota(jnp.int32, 16)
    bufs = (buf0, buf1, buf2)
    sems = (sem0, sem1, sem2)

    def gsrc(k):
        tbl = de_hbm if k < _NCHUNK else ue_hbm
        return tbl.at[idx_v[k % _NCHUNK, :]]

    @pl.loop(0, _TPW)
    def _token(t):
        n = base + t
        cx = pltpu.async_copy(x_hbm.at[n], x_v, sem0)
        ci = pltpu.async_copy(idx_hbm.at[n], idx_v, sem1)
        cr = pltpu.async_copy(rw_hbm.at[n], rw_v, sem2)
        cx.wait()
        ci.wait()
        cr.wait()

        pltpu.async_copy(gsrc(0), bufs[0], sems[0])
        pltpu.async_copy(gsrc(1), bufs[1], sems[1])
        for k in range(2 * _NCHUNK):
            if k + 2 < 2 * _NCHUNK:
                pltpu.async_copy(gsrc(k + 2), bufs[(k + 2) % 3],
                                 sems[(k + 2) % 3])
            pltpu.make_async_copy(gsrc(k), bufs[k % 3], sems[k % 3]).wait()
            rows = bufs[k % 3]
            c = k % _NCHUNK
            if k < _NCHUNK:
                zero16 = tuple(jnp.zeros((16,), jnp.float32)
                               for _ in range(_EC))

                @plsc.parallel_loop(0, _DV, carry=zero16)
                def _dt(i, ps, _rows=rows):
                    xv = x_v[pl.ds(i * 16, 16)]
                    return tuple(
                        ps[e] + _rows[e, pl.ds(i * 16, 16)] * xv
                        for e in range(_EC))

                for e in range(_EC):
                    pmat_v[e, :] = _dt[e]

                ew = jnp.zeros((16,), jnp.float32)
                for d in range(16):
                    ew = ew + plsc.load_gather(
                        pmat_v, [lanes, jnp.full((16,), d, jnp.int32)])
                sig = 1.0 / (1.0 + jnp.exp(-ew))
                w_v[pl.ds(c * _EC, _EC)] = ew * sig * rw_v[c, :]
            else:
                for e in range(_EC):
                    wsp = plsc.load_gather(
                        w_v, [jnp.full((16,), c * _EC + e, jnp.int32)])
                    if c == 0 and e == 0:
                        @plsc.parallel_loop(0, _DV, unroll=8)
                        def _acc0(i, _rows=rows, _w=wsp):
                            acc_v[pl.ds(i * 16, 16)] = (
                                _w * _rows[0, pl.ds(i * 16, 16)])
                    else:
                        @plsc.parallel_loop(0, _DV, unroll=8)
                        def _accn(i, _e=e, _rows=rows, _w=wsp):
                            plsc.addupdate(acc_v.at[pl.ds(i * 16, 16)],
                                           _w * _rows[_e, pl.ds(i * 16, 16)])

        pltpu.sync_copy(acc_v, out_hbm.at[n])


def _sc_experts(down_embed, up_embed, x, indices, routing_weights):
    idx = indices.reshape(_N_TOK, _NCHUNK, _EC).astype(jnp.int32)
    rw = routing_weights.reshape(_N_TOK, _NCHUNK, _EC)
    mesh = plsc.VectorSubcoreMesh(core_axis_name="c", subcore_axis_name="s",
                                  num_cores=2, num_subcores=16)
    f = pl.kernel(
        _sc_expert_body,
        out_type=jax.ShapeDtypeStruct((_N_TOK, _HIDDEN), jnp.float32),
        mesh=mesh,
        compiler_params=pltpu.CompilerParams(needs_layout_passes=False),
        scratch_types=[
            pltpu.VMEM((_HIDDEN,), jnp.float32),
            pltpu.VMEM((_NCHUNK, _EC), jnp.int32),
            pltpu.VMEM((_NCHUNK, _EC), jnp.float32),
            pltpu.VMEM((_EC, 16), jnp.float32),
            pltpu.VMEM((_TOP_K,), jnp.float32),
            pltpu.VMEM((_HIDDEN,), jnp.float32),
            pltpu.VMEM((_EC, _HIDDEN), jnp.float32),
            pltpu.VMEM((_EC, _HIDDEN), jnp.float32),
            pltpu.VMEM((_EC, _HIDDEN), jnp.float32),
            pltpu.SemaphoreType.DMA,
            pltpu.SemaphoreType.DMA,
            pltpu.SemaphoreType.DMA,
        ],
    )
    return f(down_embed, up_embed, x, idx, rw)


def kernel(hidden_states, w_router, w_gate, w_up, w_down, down_embed, up_embed):
    bsz, seq_len, hidden_dim = hidden_states.shape
    x = hidden_states.reshape(-1, hidden_dim)
    n_tok = x.shape[0]

    logits = _router_logits(x, w_router)
    router_logits = logits.reshape(2, n_tok, _NUM_KEYS)

    sx, ix = jax.lax.top_k(router_logits[0], _TOP_K)
    sy, iy = jax.lax.top_k(router_logits[1], _TOP_K)

    cand = sx[:, _CAND_I] + sy[:, _CAND_J]
    scores, pos = jax.lax.top_k(cand, _TOP_K)
    ci = jnp.asarray(_CAND_I)[pos]
    cj = jnp.asarray(_CAND_J)[pos]
    xi = jnp.take_along_axis(ix, ci, axis=1)
    yi = jnp.take_along_axis(iy, cj, axis=1)
    indices = xi * _NUM_KEYS + yi
    routing_weights = jax.nn.softmax(scores, axis=-1)

    experts_states = _sc_experts(down_embed, up_embed, x, indices,
                                 routing_weights)
    shared = _shared_mlp(x, w_gate, w_up, w_down)

    out = shared + experts_states
    return out.reshape(bsz, seq_len, -1), router_logits


# speedup vs baseline: 10.3699x; 2.6475x over previous
import functools

import jax
import jax.numpy as jnp
import numpy as np
from jax import lax
from jax.experimental import pallas as pl
from jax.experimental.pallas import tpu as pltpu
from jax.experimental.pallas import tpu_sc as plsc

_HIDDEN = 2048
_INTER = 4096
_NUM_KEYS = 128
_TOP_K = 64
_IB = 256
_N_TOK = 2048
_NW = 32
_TPW = _N_TOK // _NW
_EC = 16
_NCHUNK = _TOP_K // _EC
_DV = _HIDDEN // 16


def _candidate_pairs():
    pairs = [(i, j) for i in range(_TOP_K) for j in range(_TOP_K)
             if (i + 1) * (j + 1) <= _TOP_K]
    ci = np.array([p[0] for p in pairs], np.int32)
    cj = np.array([p[1] for p in pairs], np.int32)
    return ci, cj


_CAND_I, _CAND_J = _candidate_pairs()


def _router_body(x_ref, wr_ref, logits_ref, lt_ref):
    logits_ref[...] = jnp.dot(x_ref[...], wr_ref[...],
                              preferred_element_type=jnp.float32)
    lt_ref[...] = jax.lax.dot_general(
        wr_ref[...], x_ref[...], (((0,), (1,)), ((), ())),
        preferred_element_type=jnp.float32)


def _router_logits(x, w_router):
    n = x.shape[0]
    return pl.pallas_call(
        _router_body,
        out_shape=[
            jax.ShapeDtypeStruct((n, 2 * _NUM_KEYS), jnp.float32),
            jax.ShapeDtypeStruct((2 * _NUM_KEYS, n), jnp.float32),
        ],
    )(x, w_router)


def _mlp_body(x_ref, wg_ref, wu_ref, wd_ref, shared_ref):
    i = pl.program_id(0)

    @pl.when(i == 0)
    def _():
        shared_ref[...] = jnp.zeros_like(shared_ref)

    xb = x_ref[...]
    g = jnp.dot(xb, wg_ref[...], preferred_element_type=jnp.float32)
    u = jnp.dot(xb, wu_ref[...], preferred_element_type=jnp.float32)
    h = jax.nn.silu(g) * u
    shared_ref[...] += jnp.dot(h, wd_ref[...],
                               preferred_element_type=jnp.float32)


def _shared_mlp(x, w_gate, w_up, w_down):
    n = x.shape[0]
    ni = _INTER // _IB
    return pl.pallas_call(
        _mlp_body,
        grid=(ni,),
        in_specs=[
            pl.BlockSpec((n, _HIDDEN), lambda i: (0, 0)),
            pl.BlockSpec((_HIDDEN, _IB), lambda i: (0, i)),
            pl.BlockSpec((_HIDDEN, _IB), lambda i: (0, i)),
            pl.BlockSpec((_IB, _HIDDEN), lambda i: (i, 0)),
        ],
        out_specs=pl.BlockSpec((n, _HIDDEN), lambda i: (0, 0)),
        out_shape=jax.ShapeDtypeStruct((n, _HIDDEN), jnp.float32),
    )(x, w_gate, w_up, w_down)


_TB = 512
_NCAND = len(_CAND_I)


def _routing_consts():
    perms = np.zeros((7, 128, 128), np.float32)
    p = np.arange(128)
    for dlog in range(7):
        perms[dlog, p, p ^ (1 << dlog)] = 1.0
    cv = np.zeros((512, 256), np.float32)
    cp = np.zeros((512, 256), np.float32)
    for c, (i, j) in enumerate(zip(_CAND_I, _CAND_J)):
        cv[c, i] = 1.0
        cv[c, 128 + j] = 1.0
        cp[c, i] = 128.0
        cp[c, 128 + j] = 1.0
    return perms, cv, cp


_PERMS, _CV, _CP = _routing_consts()


def _topk_body(s_ref, p7_ref, cv_ref, cp_ref, w_ref, i_ref):
    io1 = lax.broadcasted_iota(jnp.int32, (128, 1), 0)

    def cmpex(vb, ib, dlog, m):
        pm = p7_ref[dlog]
        pv = jnp.dot(pm, vb, preferred_element_type=jnp.float32)
        pi = jnp.dot(pm, ib, preferred_element_type=jnp.float32)
        take = (vb > pv) == m
        return jnp.where(take, vb, pv), jnp.where(take, ib, pi)

    v0, v1 = s_ref[0:128, :], s_ref[128:256, :]
    i0 = lax.broadcasted_iota(jnp.int32, (128, _TB), 0).astype(jnp.float32)
    i1 = i0
    for s_log in range(1, 8):
        for d_log in reversed(range(s_log)):
            d = 1 << d_log
            m = ((io1 & d) == 0) == (((io1 >> s_log) & 1) == 0)
            v0, i0 = cmpex(v0, i0, d_log, m)
            v1, i1 = cmpex(v1, i1, d_log, m)

    sv = jnp.concatenate([v0, v1], axis=0)
    si = jnp.concatenate([i0, i1], axis=0)
    blocks = []
    for g in range(4):
        cvg = cv_ref[pl.ds(g * 128, 128), :]
        cpg = cp_ref[pl.ds(g * 128, 128), :]
        vb = jnp.dot(cvg, sv, preferred_element_type=jnp.float32)
        ib = jnp.dot(cpg, si, preferred_element_type=jnp.float32)
        valid = (io1 + g * 128) < _NCAND
        vb = jnp.where(valid, vb, -1e30)
        blocks.append((vb, ib))

    for s_log in range(1, 7):
        for d_log in reversed(range(s_log)):
            d = 1 << d_log
            m = ((io1 & d) == 0) == (((io1 >> s_log) & 1) == 0)
            blocks = [cmpex(vb, ib, d_log, m) for vb, ib in blocks]

    mmask = (io1 & 64) == 0
    for level in range(3):
        halves = []
        for vb, ib in blocks:
            vb, ib = cmpex(vb, ib, 6, mmask)
            halves.append((vb[0:64, :], ib[0:64, :]))
        blocks = []
        for h in range(0, len(halves), 2):
            if h + 1 < len(halves):
                vb = jnp.concatenate([halves[h][0], halves[h + 1][0]], axis=0)
                ib = jnp.concatenate([halves[h][1], halves[h + 1][1]], axis=0)
                blocks.append((vb, ib))
            else:
                blocks.append(halves[h])
        if len(blocks) == 1 and blocks[0][0].shape[0] == 64:
            break
        for d_log in reversed(range(6)):
            d = 1 << d_log
            m = ((io1 & d) == 0) == (((io1 >> 6) & 1) == 0)
            blocks = [cmpex(vb, ib, d_log, m) for vb, ib in blocks]

    vtop, itop = blocks[0]
    mx = jnp.max(vtop, axis=0, keepdims=True)
    e = jnp.exp(vtop - mx)
    w_ref[...] = e / jnp.sum(e, axis=0, keepdims=True)
    i_ref[...] = itop.astype(jnp.int32)


def _routing_topk(s_stack):
    n = s_stack.shape[1]
    nb = n // _TB
    return pl.pallas_call(
        _topk_body,
        grid=(nb,),
        in_specs=[
            pl.BlockSpec((2 * _NUM_KEYS, _TB), lambda b: (0, b)),
            pl.BlockSpec((7, 128, 128), lambda b: (0, 0, 0)),
            pl.BlockSpec((512, 256), lambda b: (0, 0)),
            pl.BlockSpec((512, 256), lambda b: (0, 0)),
        ],
        out_specs=[
            pl.BlockSpec((_TOP_K, _TB), lambda b: (0, b)),
            pl.BlockSpec((_TOP_K, _TB), lambda b: (0, b)),
        ],
        out_shape=[
            jax.ShapeDtypeStruct((_TOP_K, n), jnp.float32),
            jax.ShapeDtypeStruct((_TOP_K, n), jnp.int32),
        ],
    )(s_stack, jnp.asarray(_PERMS), jnp.asarray(_CV), jnp.asarray(_CP))


def _sc_expert_body(de_hbm, ue_hbm, x_hbm, idx_hbm, rw_hbm, out_hbm,
                    x_v, idx_v, rw_v, pmat_v, w_v, acc_v,
                    buf0, buf1, buf2, sem0, sem1, sem2):
    wid = lax.axis_index("s") * 2 + lax.axis_index("c")
    base = wid * _TPW
    lanes = lax.iota(jnp.int32, 16)
    bufs = (buf0, buf1, buf2)
    sems = (sem0, sem1, sem2)

    def gsrc(k):
        tbl = de_hbm if k < _NCHUNK else ue_hbm
        return tbl.at[idx_v[k % _NCHUNK, :]]

    @pl.loop(0, _TPW)
    def _token(t):
        n = base + t
        cx = pltpu.async_copy(x_hbm.at[n], x_v, sem0)
        ci = pltpu.async_copy(idx_hbm.at[n], idx_v, sem1)
        cr = pltpu.async_copy(rw_hbm.at[n], rw_v, sem2)
        cx.wait()
        ci.wait()
        cr.wait()

        pltpu.async_copy(gsrc(0), bufs[0], sems[0])
        pltpu.async_copy(gsrc(1), bufs[1], sems[1])
        for k in range(2 * _NCHUNK):
            if k + 2 < 2 * _NCHUNK:
                pltpu.async_copy(gsrc(k + 2), bufs[(k + 2) % 3],
                                 sems[(k + 2) % 3])
            pltpu.make_async_copy(gsrc(k), bufs[k % 3], sems[k % 3]).wait()
            rows = bufs[k % 3]
            c = k % _NCHUNK
            if k < _NCHUNK:
                zero16 = tuple(jnp.zeros((16,), jnp.float32)
                               for _ in range(_EC))

                @plsc.parallel_loop(0, _DV, carry=zero16)
                def _dt(i, ps, _rows=rows):
                    xv = x_v[pl.ds(i * 16, 16)]
                    return tuple(
                        ps[e] + _rows[e, pl.ds(i * 16, 16)] * xv
                        for e in range(_EC))

                for e in range(_EC):
                    pmat_v[e, :] = _dt[e]

                ew = jnp.zeros((16,), jnp.float32)
                for d in range(16):
                    ew = ew + plsc.load_gather(
                        pmat_v, [lanes, jnp.full((16,), d, jnp.int32)])
                sig = 1.0 / (1.0 + jnp.exp(-ew))
                w_v[pl.ds(c * _EC, _EC)] = ew * sig * rw_v[c, :]
            else:
                for e in range(_EC):
                    wsp = plsc.load_gather(
                        w_v, [jnp.full((16,), c * _EC + e, jnp.int32)])
                    if c == 0 and e == 0:
                        @plsc.parallel_loop(0, _DV, unroll=8)
                        def _acc0(i, _rows=rows, _w=wsp):
                            acc_v[pl.ds(i * 16, 16)] = (
                                _w * _rows[0, pl.ds(i * 16, 16)])
                    else:
                        @plsc.parallel_loop(0, _DV, unroll=8)
                        def _accn(i, _e=e, _rows=rows, _w=wsp):
                            plsc.addupdate(acc_v.at[pl.ds(i * 16, 16)],
                                           _w * _rows[_e, pl.ds(i * 16, 16)])

        pltpu.sync_copy(acc_v, out_hbm.at[n])


def _sc_experts(down_embed, up_embed, x, indices, routing_weights):
    idx = indices.reshape(_N_TOK, _NCHUNK, _EC).astype(jnp.int32)
    rw = routing_weights.reshape(_N_TOK, _NCHUNK, _EC)
    mesh = plsc.VectorSubcoreMesh(core_axis_name="c", subcore_axis_name="s",
                                  num_cores=2, num_subcores=16)
    f = pl.kernel(
        _sc_expert_body,
        out_type=jax.ShapeDtypeStruct((_N_TOK, _HIDDEN), jnp.float32),
        mesh=mesh,
        compiler_params=pltpu.CompilerParams(needs_layout_passes=False),
        scratch_types=[
            pltpu.VMEM((_HIDDEN,), jnp.float32),
            pltpu.VMEM((_NCHUNK, _EC), jnp.int32),
            pltpu.VMEM((_NCHUNK, _EC), jnp.float32),
            pltpu.VMEM((_EC, 16), jnp.float32),
            pltpu.VMEM((_TOP_K,), jnp.float32),
            pltpu.VMEM((_HIDDEN,), jnp.float32),
            pltpu.VMEM((_EC, _HIDDEN), jnp.float32),
            pltpu.VMEM((_EC, _HIDDEN), jnp.float32),
            pltpu.VMEM((_EC, _HIDDEN), jnp.float32),
            pltpu.SemaphoreType.DMA,
            pltpu.SemaphoreType.DMA,
            pltpu.SemaphoreType.DMA,
        ],
    )
    return f(down_embed, up_embed, x, idx, rw)


def kernel(hidden_states, w_router, w_gate, w_up, w_down, down_embed, up_embed):
    bsz, seq_len, hidden_dim = hidden_states.shape
    x = hidden_states.reshape(-1, hidden_dim)
    n_tok = x.shape[0]

    logits, lt = _router_logits(x, w_router)
    router_logits = logits.reshape(2, n_tok, _NUM_KEYS)

    half = n_tok // 2
    sxt = (lt[:, :half].reshape(2, _NUM_KEYS, half)
           .transpose(1, 2, 0).reshape(_NUM_KEYS, n_tok))
    syt = (lt[:, half:].reshape(2, _NUM_KEYS, half)
           .transpose(1, 2, 0).reshape(_NUM_KEYS, n_tok))
    w_t, i_t = _routing_topk(jnp.concatenate([sxt, syt], axis=0))
    routing_weights = w_t.T
    indices = i_t.T

    experts_states = _sc_experts(down_embed, up_embed, x, indices,
                                 routing_weights)
    shared = _shared_mlp(x, w_gate, w_up, w_down)

    out = shared + experts_states
    return out.reshape(bsz, seq_len, -1), router_logits
